# baseline (speedup 1.0000x reference)
import functools

import jax
import jax.numpy as jnp
from jax import lax
from jax.experimental import pallas as pl
from jax.experimental.pallas import tpu as pltpu
from jax.experimental.pallas import tpu_sc as plsc

B = 16384
E = 64
P = 10
N = 50
W = 64
NC = 2
NS = 16
NW = NC * NS
BPW = B // NW
CHUNK = 8
NCHUNK = BPW // CHUNK
VP = 500000


def _sc_scores_body(wpair_hbm, wpar_hbm, cent_hbm, out2_hbm, scores_hbm,
                    cent_v, idx_v, par_v, rows_v, scores_v, sem):
    wid = lax.axis_index("s") * NC + lax.axis_index("c")
    base = wid * BPW

    cb = pl.multiple_of(base // 2, 8)
    pltpu.sync_copy(cent_hbm.at[pl.ds(cb, BPW // 2), :], cent_v)

    iota = lax.iota(jnp.int32, 16)

    def chunk_body(c, carry):
        lbase = c * CHUNK
        eb = base + lbase
        pltpu.sync_copy(wpair_hbm.at[pl.ds(eb * W, CHUNK * W)], idx_v)
        pltpu.sync_copy(wpar_hbm.at[pl.ds(eb * W, CHUNK * W)], par_v)
        pltpu.async_copy(out2_hbm.at[idx_v], rows_v, sem).wait()

        def elem_body(j, inner):
            l = lbase + j
            row0 = j * W
            ch = (l & 1) * E
            vrow = [cent_v[l >> 1, pl.ds(ch + k * 16, 16)] for k in range(4)]
            for g in range(4):
                pv = par_v[pl.ds(row0 + g * 16, 16)]
                accv = jnp.zeros((16,), jnp.float32)
                for t in range(16):
                    w = g * 16 + t
                    off = pv[t] * E
                    p = None
                    for k in range(4):
                        u = rows_v[row0 + w, pl.ds(off + k * 16, 16)]
                        uk = u * vrow[k]
                        p = uk if p is None else p + uk
                    s = jnp.sum(p)
                    accv = jnp.where(iota == t, s, accv)
                srow = (c & 1) * (CHUNK // 2) + (j >> 1)
                scores_v[srow, pl.ds((j & 1) * E + g * 16, 16)] = accv
            return inner

        lax.fori_loop(0, CHUNK, elem_body, 0)

        @pl.when((c & 1) == 1)
        def _():
            sb = pl.multiple_of(base // 2 + (c - 1) * (CHUNK // 2), 8)
            pltpu.sync_copy(scores_v, scores_hbm.at[pl.ds(sb, CHUNK), :])
        return carry

    lax.fori_loop(0, NCHUNK, chunk_body, 0)


def _make_sc_scores():
    mesh = plsc.VectorSubcoreMesh(
        core_axis_name="c", subcore_axis_name="s",
        num_cores=NC, num_subcores=NS)
    return pl.kernel(
        _sc_scores_body,
        out_type=jax.ShapeDtypeStruct((B // 2, 2 * W), jnp.float32),
        mesh=mesh,
        compiler_params=pltpu.CompilerParams(
            needs_layout_passes=False, use_tc_tiling_on_sc=True),
        scratch_types=[
            pltpu.VMEM((BPW // 2, 2 * E), jnp.float32),
            pltpu.VMEM((CHUNK * W,), jnp.int32),
            pltpu.VMEM((CHUNK * W,), jnp.int32),
            pltpu.VMEM((CHUNK * W, 2 * E), jnp.float32),
            pltpu.VMEM((CHUNK, 2 * W), jnp.float32),
            pltpu.SemaphoreType.DMA,
        ],
    )


def _tc_loss_body(s_ref, o_ref):
    s = s_ref[...]
    col = lax.broadcasted_iota(jnp.int32, s.shape, 1) & (W - 1)
    x = jnp.where(col < P, s, -s)
    ls = jax.nn.log_sigmoid(x)
    ls = jnp.where(col < P + N, ls, 0.0)
    o_ref[...] = -jnp.stack(
        [jnp.sum(ls[:, :W], axis=1), jnp.sum(ls[:, W:], axis=1)], axis=1)


def _tc_loss(scores):
    blk = 2048
    out = pl.pallas_call(
        _tc_loss_body,
        grid=(B // 2 // blk,),
        in_specs=[pl.BlockSpec((blk, 2 * W), lambda i: (i, 0))],
        out_specs=pl.BlockSpec((blk, 2), lambda i: (i, 0)),
        out_shape=jax.ShapeDtypeStruct((B // 2, 2), jnp.float32),
    )(scores)
    return out.reshape(B)


@jax.jit
def kernel(center_word, target_word, negative_word, in_embed, out_embed):
    npad = W - P - N
    pad = (jnp.arange(B, dtype=jnp.int32)[:, None] * npad
           + jnp.arange(npad, dtype=jnp.int32)[None, :]) % (B * npad)
    wids = jnp.concatenate(
        [target_word.astype(jnp.int32),
         negative_word.astype(jnp.int32), pad], axis=1).reshape(-1)
    wpair = wids >> 1
    wpar = wids & 1
    cent = jnp.take(in_embed, center_word.astype(jnp.int32), axis=0,
                    mode="clip").reshape(B // 2, 2 * E)
    out2 = jnp.concatenate([out_embed[0::2], out_embed[1::2]], axis=1)
    scores = _make_sc_scores()(wpair, wpar, cent, out2)
    return _tc_loss(scores)


# speedup vs baseline: 8.5362x; 8.5362x over previous
import functools

import jax
import jax.numpy as jnp
from jax import lax
from jax.experimental import pallas as pl
from jax.experimental.pallas import tpu as pltpu
from jax.experimental.pallas import tpu_sc as plsc

B = 16384
E = 64
P = 10
N = 50
W = 64
NC = 2
NS = 16
NW = NC * NS
BPW = B // NW
CHUNK = 16
NCHUNK = BPW // CHUNK


def _sc_scores_body(wids_hbm, cent_hbm, out_e_hbm, scores_hbm,
                    cent_v, idx_v, rows_v, scores_v, sem):
    wid = lax.axis_index("s") * NC + lax.axis_index("c")
    base = wid * BPW

    pltpu.sync_copy(cent_hbm.at[pl.ds(base, BPW), :], cent_v)

    iota = lax.iota(jnp.int32, 16)

    def chunk_body(c, carry):
        lbase = c * CHUNK
        eb = base + lbase
        pltpu.sync_copy(wids_hbm.at[pl.ds(eb * W, CHUNK * W)], idx_v)
        pltpu.async_copy(out_e_hbm.at[idx_v], rows_v, sem).wait()

        def elem_body(j, inner):
            l = lbase + j
            row0 = j * W
            vrow = [cent_v[l, pl.ds(k * 16, 16)] for k in range(4)]
            for g in range(4):
                accv = jnp.zeros((16,), jnp.float32)
                for t in range(16):
                    w = g * 16 + t
                    p = None
                    for k in range(4):
                        u = rows_v[row0 + w, pl.ds(k * 16, 16)]
                        uk = u * vrow[k]
                        p = uk if p is None else p + uk
                    s = jnp.sum(p)
                    accv = jnp.where(iota == t, s, accv)
                scores_v[j, pl.ds(g * 16, 16)] = accv
            return inner

        lax.fori_loop(0, CHUNK, elem_body, 0)
        pltpu.sync_copy(scores_v, scores_hbm.at[pl.ds(eb, CHUNK), :])
        return carry

    lax.fori_loop(0, NCHUNK, chunk_body, 0)


def _make_sc_scores():
    mesh = plsc.VectorSubcoreMesh(
        core_axis_name="c", subcore_axis_name="s",
        num_cores=NC, num_subcores=NS)
    return pl.kernel(
        _sc_scores_body,
        out_type=jax.ShapeDtypeStruct((B, W), jnp.float32),
        mesh=mesh,
        compiler_params=pltpu.CompilerParams(
            needs_layout_passes=False, use_tc_tiling_on_sc=False),
        scratch_types=[
            pltpu.VMEM((BPW, E), jnp.float32),
            pltpu.VMEM((CHUNK * W,), jnp.int32),
            pltpu.VMEM((CHUNK * W, E), jnp.float32),
            pltpu.VMEM((CHUNK, W), jnp.float32),
            pltpu.SemaphoreType.DMA,
        ],
    )


def _tc_loss_body(s_ref, o_ref):
    s = s_ref[...]
    col = lax.broadcasted_iota(jnp.int32, s.shape, 1)
    x = jnp.where(col < P, s, -s)
    ls = jax.nn.log_sigmoid(x)
    ls = jnp.where(col < P + N, ls, 0.0)
    o_ref[...] = -jnp.sum(ls, axis=1)


def _tc_loss(scores):
    blk = 2048
    return pl.pallas_call(
        _tc_loss_body,
        grid=(B // blk,),
        in_specs=[pl.BlockSpec((blk, W), lambda i: (i, 0))],
        out_specs=pl.BlockSpec((blk,), lambda i: (i,)),
        out_shape=jax.ShapeDtypeStruct((B,), jnp.float32),
    )(scores)


@jax.jit
def kernel(center_word, target_word, negative_word, in_embed, out_embed):
    npad = W - P - N
    pad = (jnp.arange(B, dtype=jnp.int32)[:, None] * npad
           + jnp.arange(npad, dtype=jnp.int32)[None, :]) % (B * npad)
    wids = jnp.concatenate(
        [target_word.astype(jnp.int32),
         negative_word.astype(jnp.int32), pad], axis=1).reshape(-1)
    cent = jnp.take(in_embed, center_word.astype(jnp.int32), axis=0,
                    mode="clip")
    scores = _make_sc_scores()(wids, cent, out_embed)
    return _tc_loss(scores)


# speedup vs baseline: 9.1500x; 1.0719x over previous
import functools

import jax
import jax.numpy as jnp
from jax import lax
from jax.experimental import pallas as pl
from jax.experimental.pallas import tpu as pltpu
from jax.experimental.pallas import tpu_sc as plsc

B = 16384
E = 64
P = 10
N = 50
W = 64
NC = 2
NS = 16
NW = NC * NS
BPW = B // NW
CHUNK = 8
NCHUNK = BPW // CHUNK


def _sc_scores_body(wids_hbm, cent_hbm, out_e_hbm, scores_hbm,
                    cent_v, idx_v0, idx_v1, rows_v0, rows_v1, scores_v,
                    sem0, sem1):
    idx_v = (idx_v0, idx_v1)
    rows_v = (rows_v0, rows_v1)
    sem = (sem0, sem1)
    wid = lax.axis_index("s") * NC + lax.axis_index("c")
    base = wid * BPW

    pltpu.sync_copy(cent_hbm.at[pl.ds(base, BPW), :], cent_v)

    iota = lax.iota(jnp.int32, 16)

    def fire(c, q):
        eb = base + c * CHUNK
        pltpu.sync_copy(wids_hbm.at[pl.ds(eb * W, CHUNK * W)], idx_v[q])
        pltpu.async_copy(out_e_hbm.at[idx_v[q]], rows_v[q], sem[q])

    def compute(c, q):
        lbase = c * CHUNK
        eb = base + lbase
        pltpu.make_async_copy(out_e_hbm.at[idx_v[q]], rows_v[q], sem[q]).wait()

        def elem_body(j, inner):
            l = lbase + j
            row0 = j * W
            vrow = [cent_v[l, pl.ds(k * 16, 16)] for k in range(4)]
            for g in range(4):
                accv = jnp.zeros((16,), jnp.float32)
                for t in range(16):
                    w = g * 16 + t
                    p = None
                    for k in range(4):
                        u = rows_v[q][row0 + w, pl.ds(k * 16, 16)]
                        uk = u * vrow[k]
                        p = uk if p is None else p + uk
                    s = jnp.sum(p)
                    accv = jnp.where(iota == t, s, accv)
                scores_v[j, pl.ds(g * 16, 16)] = accv
            return inner

        lax.fori_loop(0, CHUNK, elem_body, 0)
        pltpu.sync_copy(scores_v, scores_hbm.at[pl.ds(eb, CHUNK), :])

    fire(0, 0)

    def pair_body(h, carry):
        c = h * 2
        fire(c + 1, 1)
        compute(c, 0)

        @pl.when(h + 1 < NCHUNK // 2)
        def _():
            fire(c + 2, 0)
        compute(c + 1, 1)
        return carry

    lax.fori_loop(0, NCHUNK // 2, pair_body, 0)


def _make_sc_scores():
    mesh = plsc.VectorSubcoreMesh(
        core_axis_name="c", subcore_axis_name="s",
        num_cores=NC, num_subcores=NS)
    return pl.kernel(
        _sc_scores_body,
        out_type=jax.ShapeDtypeStruct((B, W), jnp.float32),
        mesh=mesh,
        compiler_params=pltpu.CompilerParams(
            needs_layout_passes=False, use_tc_tiling_on_sc=False),
        scratch_types=[
            pltpu.VMEM((BPW, E), jnp.float32),
            pltpu.VMEM((CHUNK * W,), jnp.int32),
            pltpu.VMEM((CHUNK * W,), jnp.int32),
            pltpu.VMEM((CHUNK * W, E), jnp.float32),
            pltpu.VMEM((CHUNK * W, E), jnp.float32),
            pltpu.VMEM((CHUNK, W), jnp.float32),
            pltpu.SemaphoreType.DMA,
            pltpu.SemaphoreType.DMA,
        ],
    )


def _tc_loss_body(s_ref, o_ref):
    s = s_ref[...]
    col = lax.broadcasted_iota(jnp.int32, s.shape, 1)
    x = jnp.where(col < P, s, -s)
    ls = jax.nn.log_sigmoid(x)
    ls = jnp.where(col < P + N, ls, 0.0)
    o_ref[...] = -jnp.sum(ls, axis=1)


def _tc_loss(scores):
    blk = 2048
    return pl.pallas_call(
        _tc_loss_body,
        grid=(B // blk,),
        in_specs=[pl.BlockSpec((blk, W), lambda i: (i, 0))],
        out_specs=pl.BlockSpec((blk,), lambda i: (i,)),
        out_shape=jax.ShapeDtypeStruct((B,), jnp.float32),
    )(scores)


@jax.jit
def kernel(center_word, target_word, negative_word, in_embed, out_embed):
    npad = W - P - N
    pad = (jnp.arange(B, dtype=jnp.int32)[:, None] * npad
           + jnp.arange(npad, dtype=jnp.int32)[None, :]) % (B * npad)
    wids = jnp.concatenate(
        [target_word.astype(jnp.int32),
         negative_word.astype(jnp.int32), pad], axis=1).reshape(-1)
    cent = jnp.take(in_embed, center_word.astype(jnp.int32), axis=0,
                    mode="clip")
    scores = _make_sc_scores()(wids, cent, out_embed)
    return _tc_loss(scores)
